# Initial kernel scaffold; baseline (speedup 1.0000x reference)
import jax
import jax.numpy as jnp
from jax.experimental import pallas as pl


def kernel(vectors, node_feats, node_specie, radial_embedding, senders, receivers, W_skip, Wr1, br1, Wr2, br2, W_lin, w_prod, W_prodlin, W_read):
    raise NotImplementedError("write your pallas kernel here")


# speedup vs baseline: 5.2650x; 5.2650x over previous
import functools

import jax
import jax.numpy as jnp
from jax import lax
from jax.experimental import pallas as pl
from jax.experimental.pallas import tpu as pltpu
from jax.experimental.pallas import tpu_sc as plsc

_N = 10000
_E = 160000
_F = 128
_NB = 8
_SHD = 9
_NSPEC = 10
_INV_SQRT_AVG = 0.25

_NC = 2
_NS = 16
_NW = _NC * _NS
_EPW = _E // _NW
_CH = 40
_NCHUNK = _EPW // _CH
_NPAD = 10240
_NPT = _NPAD // _NS

def _sc_gather_body(nf_hbm, snd3_hbm, out_hbm, idx_v, rows_v, sem):
    c = lax.axis_index("c")
    s = lax.axis_index("s")
    wid = c * _NS + s
    base0 = wid * _EPW
    pltpu.sync_copy(snd3_hbm.at[wid], idx_v)

    def body(i, _):
        pltpu.async_copy(nf_hbm.at[idx_v.at[i]], rows_v, sem).wait()
        pltpu.sync_copy(rows_v, out_hbm.at[pl.ds(base0 + i * _CH, _CH), :])
        return ()

    lax.fori_loop(0, _NCHUNK, body, (), unroll=False)


def _sc_scatter_body(m_hbm, rcv3_hbm, zeros_hbm, out_hbm, acc_sh, idx_v,
                     rows_v, sem):
    c = lax.axis_index("c")
    s = lax.axis_index("s")
    wid = c * _NS + s
    base0 = wid * _EPW
    nbase = s * _NPT
    pltpu.sync_copy(zeros_hbm, acc_sh.at[pl.ds(nbase, _NPT), :])
    pltpu.sync_copy(rcv3_hbm.at[wid], idx_v)
    plsc.subcore_barrier()

    def body(i, _):
        pltpu.sync_copy(m_hbm.at[pl.ds(base0 + i * _CH, _CH), :], rows_v)
        pltpu.sync_copy(rows_v, acc_sh.at[idx_v.at[i]], add=True)
        return ()

    lax.fori_loop(0, _NCHUNK, body, (), unroll=False)
    plsc.subcore_barrier()
    pltpu.sync_copy(acc_sh.at[pl.ds(nbase, _NPT), :],
                    out_hbm.at[c, pl.ds(nbase, _NPT), :])


@functools.lru_cache(maxsize=None)
def _sc_impls():
    mesh = plsc.VectorSubcoreMesh(core_axis_name="c", subcore_axis_name="s",
                                  num_cores=_NC, num_subcores=_NS)
    gather = pl.kernel(
        _sc_gather_body,
        out_type=jax.ShapeDtypeStruct((_E, _F), jnp.float32),
        mesh=mesh,
        scratch_types=[
            pltpu.VMEM((_NCHUNK, _CH), jnp.int32),
            pltpu.VMEM((_CH, _F), jnp.float32),
            pltpu.SemaphoreType.DMA,
        ],
    )
    scatter = pl.kernel(
        _sc_scatter_body,
        out_type=jax.ShapeDtypeStruct((_NC, _NPAD, _F), jnp.float32),
        mesh=mesh,
        scratch_types=[
            pltpu.VMEM_SHARED((_NPAD, _F), jnp.float32),
            pltpu.VMEM((_NCHUNK, _CH), jnp.int32),
            pltpu.VMEM((_CH, _F), jnp.float32),
            pltpu.SemaphoreType.DMA,
        ],
    )
    return gather, scatter


_TE = 1280


def _tc_edge_body(vec_ref, rad_ref, h_ref, wr1_ref, br1_ref, wr2_ref, br2_ref,
                  wbig_ref, m_ref):
    v = vec_ref[...]
    r = jnp.sqrt(jnp.sum(v * v, axis=1, keepdims=True)) + 1e-8
    u = v / r
    x, y, z = u[:, 0:1], u[:, 1:2], u[:, 2:3]
    rh = jnp.dot(rad_ref[...], wr1_ref[...],
                 preferred_element_type=jnp.float32) + br1_ref[...]
    rh = rh * (1.0 / (1.0 + jnp.exp(-rh)))
    rw = jnp.dot(rh, wr2_ref[...],
                 preferred_element_type=jnp.float32) + br2_ref[...]
    h = h_ref[...]
    big = jnp.dot(h, wbig_ref[...], preferred_element_type=jnp.float32)
    ys = (jnp.ones_like(x), x, y, z,
          x * y, y * z, 3.0 * z * z - 1.0, x * z, x * x - y * y)
    acc = jnp.zeros((_TE, _F), jnp.float32)
    for lm in range(_SHD):
        coef = ys[lm] * rw[:, lm:lm + 1]
        acc = acc + coef * big[:, lm * _F:(lm + 1) * _F]
    m_ref[...] = acc


def _tc_edge(vectors, radial, h_send, wr1, br1, wr2, br2, wbig):
    grid = (_E // _TE,)
    return pl.pallas_call(
        _tc_edge_body,
        grid=grid,
        in_specs=[
            pl.BlockSpec((_TE, 3), lambda i: (i, 0)),
            pl.BlockSpec((_TE, _NB), lambda i: (i, 0)),
            pl.BlockSpec((_TE, _F), lambda i: (i, 0)),
            pl.BlockSpec((_NB, 64), lambda i: (0, 0)),
            pl.BlockSpec((1, 64), lambda i: (0, 0)),
            pl.BlockSpec((64, _SHD), lambda i: (0, 0)),
            pl.BlockSpec((1, _SHD), lambda i: (0, 0)),
            pl.BlockSpec((_F, _SHD * _F), lambda i: (0, 0)),
        ],
        out_specs=pl.BlockSpec((_TE, _F), lambda i: (i, 0)),
        out_shape=jax.ShapeDtypeStruct((_E, _F), jnp.float32),
    )(vectors, radial, h_send, wr1, br1, wr2, br2, wbig)


_TN = 1000


def _tc_node_body(p0_ref, p1_ref, nf_ref, spec_ref, wskip_ref, wprod_ref,
                  wpl_ref, wread_ref, out1_ref, feats_ref):
    f = (p0_ref[...] + p1_ref[...]) * _INV_SQRT_AVG
    spec = spec_ref[...]
    nf = nf_ref[...]
    parts = [jnp.where(spec == s, nf, 0.0) for s in range(_NSPEC)]
    xcat = jnp.concatenate(parts, axis=1)
    sc = jnp.dot(xcat, wskip_ref[...], preferred_element_type=jnp.float32)
    iota = lax.broadcasted_iota(jnp.int32, (1, _NSPEC), 1)
    onehot = (spec == iota).astype(jnp.float32)
    w = jnp.dot(onehot, wprod_ref[...], preferred_element_type=jnp.float32)
    w0, w1, w2 = w[:, :_F], w[:, _F:2 * _F], w[:, 2 * _F:3 * _F]
    pb = (w0 + w1 * f + w2 * (f * f)) * f
    feats = jnp.dot(pb, wpl_ref[...], preferred_element_type=jnp.float32) + sc
    feats_ref[...] = feats
    out1_ref[...] = jnp.dot(feats, wread_ref[...],
                            preferred_element_type=jnp.float32)


def _tc_node(p0, p1, node_feats, spec2, wskip_flat, wprod2, wpl, wread):
    grid = (_N // _TN,)
    return pl.pallas_call(
        _tc_node_body,
        grid=grid,
        in_specs=[
            pl.BlockSpec((_TN, _F), lambda i: (i, 0)),
            pl.BlockSpec((_TN, _F), lambda i: (i, 0)),
            pl.BlockSpec((_TN, _F), lambda i: (i, 0)),
            pl.BlockSpec((_TN, 1), lambda i: (i, 0)),
            pl.BlockSpec((_NSPEC * _F, _F), lambda i: (0, 0)),
            pl.BlockSpec((_NSPEC, 3 * _F), lambda i: (0, 0)),
            pl.BlockSpec((_F, _F), lambda i: (0, 0)),
            pl.BlockSpec((_F, 1), lambda i: (0, 0)),
        ],
        out_specs=[
            pl.BlockSpec((_TN, 1), lambda i: (i, 0)),
            pl.BlockSpec((_TN, _F), lambda i: (i, 0)),
        ],
        out_shape=[
            jax.ShapeDtypeStruct((_N, 1), jnp.float32),
            jax.ShapeDtypeStruct((_N, _F), jnp.float32),
        ],
    )(p0, p1, node_feats, spec2, wskip_flat, wprod2, wpl, wread)


def kernel(vectors, node_feats, node_specie, radial_embedding, senders,
           receivers, W_skip, Wr1, br1, Wr2, br2, W_lin, w_prod, W_prodlin,
           W_read):
    snd3 = senders.astype(jnp.int32).reshape(_NW, _NCHUNK, _CH)
    rcv3 = receivers.astype(jnp.int32).reshape(_NW, _NCHUNK, _CH)
    wbig = W_lin.reshape(_SHD, _F, _F).transpose(1, 0, 2).reshape(_F, _SHD * _F)
    zeros_tile = jnp.zeros((_NPT, _F), jnp.float32)

    sc_gather, sc_scatter = _sc_impls()
    h_send = sc_gather(node_feats, snd3)
    m = _tc_edge(vectors, radial_embedding, h_send, Wr1,
                 br1.reshape(1, 64), Wr2, br2.reshape(1, _SHD), wbig)
    partials = sc_scatter(m, rcv3, zeros_tile)
    p0 = partials[0, :_N]
    p1 = partials[1, :_N]

    spec2 = node_specie.astype(jnp.int32).reshape(_N, 1)
    wskip_flat = W_skip.reshape(_NSPEC * _F, _F)
    wprod2 = w_prod.reshape(_NSPEC, 3 * _F)
    node_outputs, feats = _tc_node(p0, p1, node_feats, spec2,
                                   wskip_flat, wprod2, W_prodlin, W_read)
    return node_outputs, feats


# speedup vs baseline: 5.2691x; 1.0008x over previous
import functools

import jax
import jax.numpy as jnp
from jax import lax
from jax.experimental import pallas as pl
from jax.experimental.pallas import tpu as pltpu
from jax.experimental.pallas import tpu_sc as plsc

_N = 10000
_E = 160000
_F = 128
_NB = 8
_SHD = 9
_NSPEC = 10
_INV_SQRT_AVG = 0.25

_NC = 2
_NS = 16
_NW = _NC * _NS
_EPW = _E // _NW
_CH = 40
_NCHUNK = _EPW // _CH
_NPAD = 10240
_NPT = _NPAD // _NS

def _sc_gather_body(nf_hbm, snd3_hbm, out_hbm, idx_v, rows_v, sem):
    c = lax.axis_index("c")
    s = lax.axis_index("s")
    wid = c * _NS + s
    base0 = wid * _EPW
    pltpu.sync_copy(snd3_hbm.at[wid], idx_v)

    def body(i, _):
        pltpu.async_copy(nf_hbm.at[idx_v.at[i]], rows_v, sem).wait()
        pltpu.sync_copy(rows_v, out_hbm.at[pl.ds(base0 + i * _CH, _CH), :])
        return ()

    lax.fori_loop(0, _NCHUNK, body, (), unroll=False)


def _sc_scatter_body(m_hbm, rcv3_hbm, zeros_hbm, out_hbm, acc_sh, idx_v,
                     rows_v, sem):
    c = lax.axis_index("c")
    s = lax.axis_index("s")
    wid = c * _NS + s
    base0 = wid * _EPW
    nbase = s * _NPT
    pltpu.sync_copy(zeros_hbm, acc_sh.at[pl.ds(nbase, _NPT), :])
    pltpu.sync_copy(rcv3_hbm.at[wid], idx_v)
    plsc.subcore_barrier()

    def body(i, _):
        pltpu.sync_copy(m_hbm.at[pl.ds(base0 + i * _CH, _CH), :], rows_v)
        pltpu.sync_copy(rows_v, acc_sh.at[idx_v.at[i]], add=True)
        return ()

    lax.fori_loop(0, _NCHUNK, body, (), unroll=False)
    plsc.subcore_barrier()
    pltpu.sync_copy(acc_sh.at[pl.ds(nbase, _NPT), :],
                    out_hbm.at[c, pl.ds(nbase, _NPT), :])


@functools.lru_cache(maxsize=None)
def _sc_impls():
    mesh = plsc.VectorSubcoreMesh(core_axis_name="c", subcore_axis_name="s",
                                  num_cores=_NC, num_subcores=_NS)
    gather = pl.kernel(
        _sc_gather_body,
        out_type=jax.ShapeDtypeStruct((_E, _F), jnp.float32),
        mesh=mesh,
        scratch_types=[
            pltpu.VMEM((_NCHUNK, _CH), jnp.int32),
            pltpu.VMEM((_CH, _F), jnp.float32),
            pltpu.SemaphoreType.DMA,
        ],
    )
    scatter = pl.kernel(
        _sc_scatter_body,
        out_type=jax.ShapeDtypeStruct((_NC, _NPAD, _F), jnp.float32),
        mesh=mesh,
        scratch_types=[
            pltpu.VMEM_SHARED((_NPAD, _F), jnp.float32),
            pltpu.VMEM((_NCHUNK, _CH), jnp.int32),
            pltpu.VMEM((_CH, _F), jnp.float32),
            pltpu.SemaphoreType.DMA,
        ],
    )
    return gather, scatter


_TE = 1280


def _tc_edge_body(vec_ref, rad_ref, h_ref, wr1_ref, br1_ref, wr2_ref, br2_ref,
                  wbig_ref, m_ref):
    v = vec_ref[...]
    r = jnp.sqrt(jnp.sum(v * v, axis=1, keepdims=True)) + 1e-8
    u = v / r
    x, y, z = u[:, 0:1], u[:, 1:2], u[:, 2:3]
    rh = jnp.dot(rad_ref[...], wr1_ref[...],
                 preferred_element_type=jnp.float32) + br1_ref[...]
    rh = rh * (1.0 / (1.0 + jnp.exp(-rh)))
    rw = jnp.dot(rh, wr2_ref[...],
                 preferred_element_type=jnp.float32) + br2_ref[...]
    h = h_ref[...].astype(jnp.bfloat16)
    big = jnp.dot(h, wbig_ref[...], preferred_element_type=jnp.float32)
    ys = (jnp.ones_like(x), x, y, z,
          x * y, y * z, 3.0 * z * z - 1.0, x * z, x * x - y * y)
    acc = jnp.zeros((_TE, _F), jnp.float32)
    for lm in range(_SHD):
        coef = ys[lm] * rw[:, lm:lm + 1]
        acc = acc + coef * big[:, lm * _F:(lm + 1) * _F]
    m_ref[...] = acc


def _tc_edge(vectors, radial, h_send, wr1, br1, wr2, br2, wbig):
    grid = (_E // _TE,)
    return pl.pallas_call(
        _tc_edge_body,
        grid=grid,
        in_specs=[
            pl.BlockSpec((_TE, 3), lambda i: (i, 0)),
            pl.BlockSpec((_TE, _NB), lambda i: (i, 0)),
            pl.BlockSpec((_TE, _F), lambda i: (i, 0)),
            pl.BlockSpec((_NB, 64), lambda i: (0, 0)),
            pl.BlockSpec((1, 64), lambda i: (0, 0)),
            pl.BlockSpec((64, _SHD), lambda i: (0, 0)),
            pl.BlockSpec((1, _SHD), lambda i: (0, 0)),
            pl.BlockSpec((_F, _SHD * _F), lambda i: (0, 0)),
        ],
        out_specs=pl.BlockSpec((_TE, _F), lambda i: (i, 0)),
        out_shape=jax.ShapeDtypeStruct((_E, _F), jnp.float32),
    )(vectors, radial, h_send, wr1, br1, wr2, br2, wbig)


_TN = 1000


def _tc_node_body(p0_ref, p1_ref, nf_ref, spec_ref, wskip_ref, wprod_ref,
                  wpl_ref, wread_ref, out1_ref, feats_ref):
    f = (p0_ref[...] + p1_ref[...]) * _INV_SQRT_AVG
    spec = spec_ref[...]
    nf = nf_ref[...]
    parts = [jnp.where(spec == s, nf, 0.0) for s in range(_NSPEC)]
    xcat = jnp.concatenate(parts, axis=1)
    sc = jnp.dot(xcat, wskip_ref[...], preferred_element_type=jnp.float32)
    iota = lax.broadcasted_iota(jnp.int32, (1, _NSPEC), 1)
    onehot = (spec == iota).astype(jnp.float32)
    w = jnp.dot(onehot, wprod_ref[...], preferred_element_type=jnp.float32)
    w0, w1, w2 = w[:, :_F], w[:, _F:2 * _F], w[:, 2 * _F:3 * _F]
    pb = (w0 + w1 * f + w2 * (f * f)) * f
    feats = jnp.dot(pb, wpl_ref[...], preferred_element_type=jnp.float32) + sc
    feats_ref[...] = feats
    out1_ref[...] = jnp.dot(feats, wread_ref[...],
                            preferred_element_type=jnp.float32)


def _tc_node(p0, p1, node_feats, spec2, wskip_flat, wprod2, wpl, wread):
    grid = (_N // _TN,)
    return pl.pallas_call(
        _tc_node_body,
        grid=grid,
        in_specs=[
            pl.BlockSpec((_TN, _F), lambda i: (i, 0)),
            pl.BlockSpec((_TN, _F), lambda i: (i, 0)),
            pl.BlockSpec((_TN, _F), lambda i: (i, 0)),
            pl.BlockSpec((_TN, 1), lambda i: (i, 0)),
            pl.BlockSpec((_NSPEC * _F, _F), lambda i: (0, 0)),
            pl.BlockSpec((_NSPEC, 3 * _F), lambda i: (0, 0)),
            pl.BlockSpec((_F, _F), lambda i: (0, 0)),
            pl.BlockSpec((_F, 1), lambda i: (0, 0)),
        ],
        out_specs=[
            pl.BlockSpec((_TN, 1), lambda i: (i, 0)),
            pl.BlockSpec((_TN, _F), lambda i: (i, 0)),
        ],
        out_shape=[
            jax.ShapeDtypeStruct((_N, 1), jnp.float32),
            jax.ShapeDtypeStruct((_N, _F), jnp.float32),
        ],
    )(p0, p1, node_feats, spec2, wskip_flat, wprod2, wpl, wread)


def kernel(vectors, node_feats, node_specie, radial_embedding, senders,
           receivers, W_skip, Wr1, br1, Wr2, br2, W_lin, w_prod, W_prodlin,
           W_read):
    snd3 = senders.astype(jnp.int32).reshape(_NW, _NCHUNK, _CH)
    rcv3 = receivers.astype(jnp.int32).reshape(_NW, _NCHUNK, _CH)
    wbig = (W_lin.reshape(_SHD, _F, _F).transpose(1, 0, 2)
            .reshape(_F, _SHD * _F).astype(jnp.bfloat16))
    zeros_tile = jnp.zeros((_NPT, _F), jnp.float32)

    sc_gather, sc_scatter = _sc_impls()
    h_send = sc_gather(node_feats, snd3)
    m = _tc_edge(vectors, radial_embedding, h_send, Wr1,
                 br1.reshape(1, 64), Wr2, br2.reshape(1, _SHD), wbig)
    partials = sc_scatter(m, rcv3, zeros_tile)
    p0 = partials[0, :_N]
    p1 = partials[1, :_N]

    spec2 = node_specie.astype(jnp.int32).reshape(_N, 1)
    wskip_flat = W_skip.reshape(_NSPEC * _F, _F)
    wprod2 = w_prod.reshape(_NSPEC, 3 * _F)
    node_outputs, feats = _tc_node(p0, p1, node_feats, spec2,
                                   wskip_flat, wprod2, W_prodlin, W_read)
    return node_outputs, feats
